# baseline (speedup 1.0000x reference)
import functools

import jax
import jax.numpy as jnp
from jax import lax
from jax.experimental import pallas as pl
from jax.experimental.pallas import tpu as pltpu
from jax.experimental.pallas import tpu_sc as plsc

B, S, D = 4, 8192, 1024
HID = 512
NUM_OUT = 8

SC_ROWS = 2048
SUB_PER_B = 8
T_SUB = SC_ROWS // SUB_PER_B
RC = 16
NCH = T_SUB // RC
NV = D // 16

TC_ROWS = S - SC_ROWS
CS = 2048
NC = TC_ROWS // CS


def _red16(v, op):
    lane = lax.iota(jnp.int32, 16)
    for k in (8, 4, 2, 1):
        v = op(v, v.at[lane ^ k].get(mode="promise_in_bounds"))
    return v


def _sc_pool_body(x_hbm, wp_hbm, acc_hbm, stats_hbm,
                  xbuf, wp_buf, acc_ref, st_ref):
    wid = lax.axis_index("s") * 2 + lax.axis_index("c")
    b = wid // SUB_PER_B
    row0 = (wid % SUB_PER_B) * T_SUB

    pltpu.sync_copy(wp_hbm, wp_buf)
    for j in range(NV):
        acc_ref[pl.ds(16 * j, 16)] = jnp.zeros((16,), jnp.float32)

    def chunk_body(k, carry):
        m_run, l_run = carry
        pltpu.sync_copy(x_hbm.at[b, pl.ds(row0 + k * RC, RC), :], xbuf)

        lane = lax.iota(jnp.int32, 16)

        def score_row(r, sco_c):
            s0 = jnp.zeros((16,), jnp.float32)
            s1 = jnp.zeros((16,), jnp.float32)
            s2 = jnp.zeros((16,), jnp.float32)
            s3 = jnp.zeros((16,), jnp.float32)
            for j in range(0, NV, 4):
                s0 = s0 + xbuf[r, pl.ds(16 * j, 16)] * wp_buf[pl.ds(16 * j, 16)]
                s1 = s1 + xbuf[r, pl.ds(16 * (j + 1), 16)] * wp_buf[pl.ds(16 * (j + 1), 16)]
                s2 = s2 + xbuf[r, pl.ds(16 * (j + 2), 16)] * wp_buf[pl.ds(16 * (j + 2), 16)]
                s3 = s3 + xbuf[r, pl.ds(16 * (j + 3), 16)] * wp_buf[pl.ds(16 * (j + 3), 16)]
            s = _red16((s0 + s1) + (s2 + s3), jnp.add)
            return jnp.where(lane == r, s, sco_c)

        sco = lax.fori_loop(0, RC, score_row, jnp.zeros((16,), jnp.float32))
        m_c = _red16(sco, jnp.maximum)
        m_new = jnp.maximum(m_run, m_c)
        alpha_v = jnp.exp(m_run - m_new)
        p_v = jnp.exp(sco - m_new)
        l_new = l_run * alpha_v + _red16(p_v, jnp.add)

        ps = [p_v[r] for r in range(RC)]

        def acc_col(j, _):
            t0 = ps[0] * xbuf[0, pl.ds(16 * j, 16)]
            t1 = ps[1] * xbuf[1, pl.ds(16 * j, 16)]
            t2 = ps[2] * xbuf[2, pl.ds(16 * j, 16)]
            t3 = ps[3] * xbuf[3, pl.ds(16 * j, 16)]
            for r in range(4, RC, 4):
                t0 = t0 + ps[r] * xbuf[r, pl.ds(16 * j, 16)]
                t1 = t1 + ps[r + 1] * xbuf[r + 1, pl.ds(16 * j, 16)]
                t2 = t2 + ps[r + 2] * xbuf[r + 2, pl.ds(16 * j, 16)]
                t3 = t3 + ps[r + 3] * xbuf[r + 3, pl.ds(16 * j, 16)]
            acc_ref[pl.ds(16 * j, 16)] = (acc_ref[pl.ds(16 * j, 16)] * alpha_v
                                          + ((t0 + t1) + (t2 + t3)))
            return 0

        lax.fori_loop(0, NV, acc_col, 0)
        return (m_new, l_new)

    m_fin, l_fin = lax.fori_loop(
        0, NCH, chunk_body,
        (jnp.full((16,), -jnp.inf, jnp.float32), jnp.zeros((16,), jnp.float32)))

    lane = lax.iota(jnp.int32, 16)
    st_ref[...] = jnp.where(lane == 0, m_fin,
                            jnp.where(lane == 1, l_fin,
                                      jnp.zeros((16,), jnp.float32)))
    pltpu.sync_copy(acc_ref, acc_hbm.at[wid])
    pltpu.sync_copy(st_ref, stats_hbm.at[wid])


def _sc_pool(x, wp_flat):
    return pl.kernel(
        _sc_pool_body,
        out_type=(jax.ShapeDtypeStruct((B * SUB_PER_B, D), jnp.float32),
                  jax.ShapeDtypeStruct((B * SUB_PER_B, 16), jnp.float32)),
        mesh=plsc.VectorSubcoreMesh(core_axis_name="c", subcore_axis_name="s"),
        scratch_types=[
            pltpu.VMEM((RC, D), jnp.float32),
            pltpu.VMEM((D,), jnp.float32),
            pltpu.VMEM((D,), jnp.float32),
            pltpu.VMEM((16,), jnp.float32),
        ],
    )(x, wp_flat)


def _tc_pool_kernel(x_ref, w_pool_ref, acc_out, m_out, l_out, m_ref, l_ref):
    b = pl.program_id(0)
    c = pl.program_id(1)

    @pl.when(c == 0)
    def _init():
        m_ref[0] = -jnp.inf
        l_ref[0] = 0.0

    x_blk = x_ref[0]
    wp_row = w_pool_ref[...].reshape(1, D)
    s = jnp.sum(x_blk * wp_row, axis=1, keepdims=True)
    m_c = jnp.max(s)
    m_prev = m_ref[0]
    m_new = jnp.maximum(m_prev, m_c)
    m_ref[0] = m_new
    alpha = jnp.exp(m_prev - m_new)
    p = jnp.exp(s - m_new)
    l_ref[0] = l_ref[0] * alpha + jnp.sum(p)
    acc_c = jnp.dot(p.T, x_blk, preferred_element_type=jnp.float32)

    @pl.when(c == 0)
    def _first():
        acc_out[pl.ds(b, 1), :] = acc_c

    @pl.when(c > 0)
    def _rest():
        acc_out[pl.ds(b, 1), :] = acc_out[pl.ds(b, 1), :] * alpha + acc_c

    @pl.when(c == NC - 1)
    def _fin():
        m_out[pl.ds(b, 1), :] = jnp.full((1, 128), m_ref[0], jnp.float32)
        l_out[pl.ds(b, 1), :] = jnp.full((1, 128), l_ref[0], jnp.float32)


def _tc_pool(x, w_pool):
    return pl.pallas_call(
        _tc_pool_kernel,
        grid=(B, NC),
        in_specs=[
            pl.BlockSpec((1, CS, D), lambda b, c: (b, c + SC_ROWS // CS, 0)),
            pl.BlockSpec((D, 1), lambda b, c: (0, 0)),
        ],
        out_specs=[
            pl.BlockSpec((B, D), lambda b, c: (0, 0)),
            pl.BlockSpec((B, 128), lambda b, c: (0, 0)),
            pl.BlockSpec((B, 128), lambda b, c: (0, 0)),
        ],
        out_shape=[
            jax.ShapeDtypeStruct((B, D), jnp.float32),
            jax.ShapeDtypeStruct((B, 128), jnp.float32),
            jax.ShapeDtypeStruct((B, 128), jnp.float32),
        ],
        scratch_shapes=[
            pltpu.SMEM((1,), jnp.float32),
            pltpu.SMEM((1,), jnp.float32),
        ],
    )(x, w_pool)


def _combine_kernel(acc_tc_ref, mtc_ref, ltc_ref, acc_sc_ref, st_sc_ref,
                    w1_ref, b1_ref, w2_ref, b2_ref, out_ref):
    msc = st_sc_ref[...][:, :, 0]
    lsc = st_sc_ref[...][:, :, 1]
    mtc = mtc_ref[...][:, 0:1]
    ltc = ltc_ref[...][:, 0:1]
    m_all = jnp.maximum(jnp.max(msc, axis=1, keepdims=True), mtc)
    sc_scale = jnp.exp(msc - m_all)
    tc_scale = jnp.exp(mtc - m_all)
    l_tot = ltc * tc_scale + jnp.sum(lsc * sc_scale, axis=1, keepdims=True)
    acc_tot = (acc_tc_ref[...] * tc_scale
               + jnp.sum(acc_sc_ref[...] * sc_scale[:, :, None], axis=1))
    pooled = acc_tot / l_tot

    h = jnp.dot(pooled, w1_ref[...], preferred_element_type=jnp.float32)
    h = jnp.maximum(h + b1_ref[...], 0.0)
    logits = jnp.dot(h, w2_ref[...], preferred_element_type=jnp.float32)
    logits = logits + b2_ref[...]

    col = jax.lax.broadcasted_iota(jnp.int32, (B, NUM_OUT), 1)
    m1 = jnp.max(logits, axis=1, keepdims=True)
    i1 = jnp.min(jnp.where(logits == m1, col, NUM_OUT), axis=1, keepdims=True)
    l2 = jnp.where(col == i1, -jnp.inf, logits)
    m2 = jnp.max(l2, axis=1, keepdims=True)
    i2 = jnp.min(jnp.where(l2 == m2, col, NUM_OUT), axis=1, keepdims=True)
    keep = (col == i1) | (col == i2)
    e = jnp.where(keep, jnp.exp(logits - m1), 0.0)
    out_ref[...] = e / jnp.sum(e, axis=1, keepdims=True)


def _combine(acc_tc, m_tc, l_tc, acc_sc, st_sc, w1, b1, w2, b2):
    return pl.pallas_call(
        _combine_kernel,
        out_shape=jax.ShapeDtypeStruct((B, NUM_OUT), jnp.float32),
    )(acc_tc, m_tc, l_tc, acc_sc, st_sc, w1, b1, w2, b2)


@jax.jit
def kernel(x, w_pool, b_pool, w1, b1, w2, b2):
    del b_pool
    wp_flat = w_pool.reshape(D)
    acc_sc, st_sc = _sc_pool(x, wp_flat)
    acc_tc, m_tc, l_tc = _tc_pool(x, w_pool)
    return _combine(acc_tc, m_tc, l_tc,
                    acc_sc.reshape(B, SUB_PER_B, D),
                    st_sc.reshape(B, SUB_PER_B, 16),
                    w1, b1.reshape(1, HID), w2, b2.reshape(1, NUM_OUT))


# speedup vs baseline: 1.3206x; 1.3206x over previous
import functools

import jax
import jax.numpy as jnp
from jax import lax
from jax.experimental import pallas as pl
from jax.experimental.pallas import tpu as pltpu
from jax.experimental.pallas import tpu_sc as plsc

B, S, D = 4, 8192, 1024
HID = 512
NUM_OUT = 8

SC_ROWS = 2048
SUB_PER_B = 8
T_SUB = SC_ROWS // SUB_PER_B
RC = 16
NCH = T_SUB // RC
NV = D // 16

TC_ROWS = S - SC_ROWS
CS = 2048
NC = TC_ROWS // CS


def _red16(v, op):
    lane = lax.iota(jnp.int32, 16)
    for k in (8, 4, 2, 1):
        v = op(v, v.at[lane ^ k].get(mode="promise_in_bounds"))
    return v


def _sc_pool_body(x_hbm, wp_hbm, acc_hbm, stats_hbm,
                  xbuf0, xbuf1, wp_buf, acc_ref, st_ref, sem0, sem1):
    wid = lax.axis_index("s") * 2 + lax.axis_index("c")
    b = wid // SUB_PER_B
    row0 = (wid % SUB_PER_B) * T_SUB

    pltpu.sync_copy(wp_hbm, wp_buf)
    for j in range(NV):
        acc_ref[pl.ds(16 * j, 16)] = jnp.zeros((16,), jnp.float32)

    def copy_in(k, buf, sem):
        return pltpu.make_async_copy(
            x_hbm.at[b, pl.ds(row0 + k * RC, RC), :], buf, sem)

    def process(xbuf, carry):
        m_run, l_run = carry
        lane = lax.iota(jnp.int32, 16)

        def score_row(r, sco_c):
            s0 = jnp.zeros((16,), jnp.float32)
            s1 = jnp.zeros((16,), jnp.float32)
            s2 = jnp.zeros((16,), jnp.float32)
            s3 = jnp.zeros((16,), jnp.float32)
            for j in range(0, NV, 4):
                s0 = s0 + xbuf[r, pl.ds(16 * j, 16)] * wp_buf[pl.ds(16 * j, 16)]
                s1 = s1 + xbuf[r, pl.ds(16 * (j + 1), 16)] * wp_buf[pl.ds(16 * (j + 1), 16)]
                s2 = s2 + xbuf[r, pl.ds(16 * (j + 2), 16)] * wp_buf[pl.ds(16 * (j + 2), 16)]
                s3 = s3 + xbuf[r, pl.ds(16 * (j + 3), 16)] * wp_buf[pl.ds(16 * (j + 3), 16)]
            s = _red16((s0 + s1) + (s2 + s3), jnp.add)
            return jnp.where(lane == r, s, sco_c)

        sco = lax.fori_loop(0, RC, score_row, jnp.zeros((16,), jnp.float32))
        m_c = _red16(sco, jnp.maximum)
        m_new = jnp.maximum(m_run, m_c)
        alpha_v = jnp.exp(m_run - m_new)
        p_v = jnp.exp(sco - m_new)
        l_new = l_run * alpha_v + _red16(p_v, jnp.add)

        ps = [p_v[r] for r in range(RC)]

        def acc_col(jj, _):
            for u in range(4):
                j = jj * 4 + u
                t0 = ps[0] * xbuf[0, pl.ds(16 * j, 16)]
                t1 = ps[1] * xbuf[1, pl.ds(16 * j, 16)]
                t2 = ps[2] * xbuf[2, pl.ds(16 * j, 16)]
                t3 = ps[3] * xbuf[3, pl.ds(16 * j, 16)]
                for r in range(4, RC, 4):
                    t0 = t0 + ps[r] * xbuf[r, pl.ds(16 * j, 16)]
                    t1 = t1 + ps[r + 1] * xbuf[r + 1, pl.ds(16 * j, 16)]
                    t2 = t2 + ps[r + 2] * xbuf[r + 2, pl.ds(16 * j, 16)]
                    t3 = t3 + ps[r + 3] * xbuf[r + 3, pl.ds(16 * j, 16)]
                acc_ref[pl.ds(16 * j, 16)] = (acc_ref[pl.ds(16 * j, 16)] * alpha_v
                                              + ((t0 + t1) + (t2 + t3)))
            return 0

        lax.fori_loop(0, NV // 4, acc_col, 0)
        return (m_new, l_new)

    copy_in(0, xbuf0, sem0).start()

    def pair_body(kk, carry):
        k0 = 2 * kk
        copy_in(k0, xbuf0, sem0).wait()
        copy_in(k0 + 1, xbuf1, sem1).start()
        carry = process(xbuf0, carry)
        copy_in(k0 + 1, xbuf1, sem1).wait()

        @pl.when(k0 + 2 < NCH)
        def _prefetch():
            copy_in(k0 + 2, xbuf0, sem0).start()

        return process(xbuf1, carry)

    m_fin, l_fin = lax.fori_loop(
        0, NCH // 2, pair_body,
        (jnp.full((16,), -jnp.inf, jnp.float32), jnp.zeros((16,), jnp.float32)))

    lane = lax.iota(jnp.int32, 16)
    st_ref[...] = jnp.where(lane == 0, m_fin,
                            jnp.where(lane == 1, l_fin,
                                      jnp.zeros((16,), jnp.float32)))
    pltpu.sync_copy(acc_ref, acc_hbm.at[wid])
    pltpu.sync_copy(st_ref, stats_hbm.at[wid])


def _sc_pool(x, wp_flat):
    return pl.kernel(
        _sc_pool_body,
        out_type=(jax.ShapeDtypeStruct((B * SUB_PER_B, D), jnp.float32),
                  jax.ShapeDtypeStruct((B * SUB_PER_B, 16), jnp.float32)),
        mesh=plsc.VectorSubcoreMesh(core_axis_name="c", subcore_axis_name="s"),
        scratch_types=[
            pltpu.VMEM((RC, D), jnp.float32),
            pltpu.VMEM((RC, D), jnp.float32),
            pltpu.VMEM((D,), jnp.float32),
            pltpu.VMEM((D,), jnp.float32),
            pltpu.VMEM((16,), jnp.float32),
            pltpu.SemaphoreType.DMA,
            pltpu.SemaphoreType.DMA,
        ],
    )(x, wp_flat)


def _tc_pool_kernel(x_ref, w_pool_ref, acc_out, m_out, l_out, m_ref, l_ref):
    b = pl.program_id(0)
    c = pl.program_id(1)

    @pl.when(c == 0)
    def _init():
        m_ref[0] = -jnp.inf
        l_ref[0] = 0.0

    x_blk = x_ref[0]
    wp_row = w_pool_ref[...].reshape(1, D)
    s = jnp.sum(x_blk * wp_row, axis=1, keepdims=True)
    m_c = jnp.max(s)
    m_prev = m_ref[0]
    m_new = jnp.maximum(m_prev, m_c)
    m_ref[0] = m_new
    alpha = jnp.exp(m_prev - m_new)
    p = jnp.exp(s - m_new)
    l_ref[0] = l_ref[0] * alpha + jnp.sum(p)
    acc_c = jnp.dot(p.T, x_blk, preferred_element_type=jnp.float32)

    @pl.when(c == 0)
    def _first():
        acc_out[pl.ds(b, 1), :] = acc_c

    @pl.when(c > 0)
    def _rest():
        acc_out[pl.ds(b, 1), :] = acc_out[pl.ds(b, 1), :] * alpha + acc_c

    @pl.when(c == NC - 1)
    def _fin():
        m_out[pl.ds(b, 1), :] = jnp.full((1, 128), m_ref[0], jnp.float32)
        l_out[pl.ds(b, 1), :] = jnp.full((1, 128), l_ref[0], jnp.float32)


def _tc_pool(x, w_pool):
    return pl.pallas_call(
        _tc_pool_kernel,
        grid=(B, NC),
        in_specs=[
            pl.BlockSpec((1, CS, D), lambda b, c: (b, c + SC_ROWS // CS, 0)),
            pl.BlockSpec((D, 1), lambda b, c: (0, 0)),
        ],
        out_specs=[
            pl.BlockSpec((B, D), lambda b, c: (0, 0)),
            pl.BlockSpec((B, 128), lambda b, c: (0, 0)),
            pl.BlockSpec((B, 128), lambda b, c: (0, 0)),
        ],
        out_shape=[
            jax.ShapeDtypeStruct((B, D), jnp.float32),
            jax.ShapeDtypeStruct((B, 128), jnp.float32),
            jax.ShapeDtypeStruct((B, 128), jnp.float32),
        ],
        scratch_shapes=[
            pltpu.SMEM((1,), jnp.float32),
            pltpu.SMEM((1,), jnp.float32),
        ],
    )(x, w_pool)


def _combine_kernel(acc_tc_ref, mtc_ref, ltc_ref, acc_sc_ref, st_sc_ref,
                    w1_ref, b1_ref, w2_ref, b2_ref, out_ref):
    msc = st_sc_ref[...][:, :, 0]
    lsc = st_sc_ref[...][:, :, 1]
    mtc = mtc_ref[...][:, 0:1]
    ltc = ltc_ref[...][:, 0:1]
    m_all = jnp.maximum(jnp.max(msc, axis=1, keepdims=True), mtc)
    sc_scale = jnp.exp(msc - m_all)
    tc_scale = jnp.exp(mtc - m_all)
    l_tot = ltc * tc_scale + jnp.sum(lsc * sc_scale, axis=1, keepdims=True)
    acc_tot = (acc_tc_ref[...] * tc_scale
               + jnp.sum(acc_sc_ref[...] * sc_scale[:, :, None], axis=1))
    pooled = acc_tot / l_tot

    h = jnp.dot(pooled, w1_ref[...], preferred_element_type=jnp.float32)
    h = jnp.maximum(h + b1_ref[...], 0.0)
    logits = jnp.dot(h, w2_ref[...], preferred_element_type=jnp.float32)
    logits = logits + b2_ref[...]

    col = jax.lax.broadcasted_iota(jnp.int32, (B, NUM_OUT), 1)
    m1 = jnp.max(logits, axis=1, keepdims=True)
    i1 = jnp.min(jnp.where(logits == m1, col, NUM_OUT), axis=1, keepdims=True)
    l2 = jnp.where(col == i1, -jnp.inf, logits)
    m2 = jnp.max(l2, axis=1, keepdims=True)
    i2 = jnp.min(jnp.where(l2 == m2, col, NUM_OUT), axis=1, keepdims=True)
    keep = (col == i1) | (col == i2)
    e = jnp.where(keep, jnp.exp(logits - m1), 0.0)
    out_ref[...] = e / jnp.sum(e, axis=1, keepdims=True)


def _combine(acc_tc, m_tc, l_tc, acc_sc, st_sc, w1, b1, w2, b2):
    return pl.pallas_call(
        _combine_kernel,
        out_shape=jax.ShapeDtypeStruct((B, NUM_OUT), jnp.float32),
    )(acc_tc, m_tc, l_tc, acc_sc, st_sc, w1, b1, w2, b2)


@jax.jit
def kernel(x, w_pool, b_pool, w1, b1, w2, b2):
    del b_pool
    wp_flat = w_pool.reshape(D)
    acc_sc, st_sc = _sc_pool(x, wp_flat)
    acc_tc, m_tc, l_tc = _tc_pool(x, w_pool)
    return _combine(acc_tc, m_tc, l_tc,
                    acc_sc.reshape(B, SUB_PER_B, D),
                    st_sc.reshape(B, SUB_PER_B, 16),
                    w1, b1.reshape(1, HID), w2, b2.reshape(1, NUM_OUT))


# speedup vs baseline: 1.3695x; 1.0370x over previous
import functools

import jax
import jax.numpy as jnp
from jax import lax
from jax.experimental import pallas as pl
from jax.experimental.pallas import tpu as pltpu
from jax.experimental.pallas import tpu_sc as plsc

B, S, D = 4, 8192, 1024
HID = 512
NUM_OUT = 8

SC_ROWS = 1024
SUB_PER_B = 8
T_SUB = SC_ROWS // SUB_PER_B
RC = 16
NCH = T_SUB // RC
NV = D // 16

TC_ROWS = S - SC_ROWS
CS = 1792
NC = TC_ROWS // CS


def _red16(v, op):
    lane = lax.iota(jnp.int32, 16)
    for k in (8, 4, 2, 1):
        v = op(v, v.at[lane ^ k].get(mode="promise_in_bounds"))
    return v


def _sc_pool_body(x_hbm, wp_hbm, acc_hbm, stats_hbm,
                  xbuf0, xbuf1, wp_buf, acc_ref, st_ref, sem0, sem1):
    wid = lax.axis_index("s") * 2 + lax.axis_index("c")
    b = wid // SUB_PER_B
    row0 = (wid % SUB_PER_B) * T_SUB

    pltpu.sync_copy(wp_hbm, wp_buf)
    for j in range(NV):
        acc_ref[pl.ds(16 * j, 16)] = jnp.zeros((16,), jnp.float32)

    def copy_in(k, buf, sem):
        return pltpu.make_async_copy(
            x_hbm.at[b, pl.ds(row0 + k * RC, RC), :], buf, sem)

    def process(xbuf, carry):
        m_run, l_run = carry
        lane = lax.iota(jnp.int32, 16)

        def score_row(r, sco_c):
            s0 = jnp.zeros((16,), jnp.float32)
            s1 = jnp.zeros((16,), jnp.float32)
            s2 = jnp.zeros((16,), jnp.float32)
            s3 = jnp.zeros((16,), jnp.float32)
            for j in range(0, NV, 4):
                s0 = s0 + xbuf[r, pl.ds(16 * j, 16)] * wp_buf[pl.ds(16 * j, 16)]
                s1 = s1 + xbuf[r, pl.ds(16 * (j + 1), 16)] * wp_buf[pl.ds(16 * (j + 1), 16)]
                s2 = s2 + xbuf[r, pl.ds(16 * (j + 2), 16)] * wp_buf[pl.ds(16 * (j + 2), 16)]
                s3 = s3 + xbuf[r, pl.ds(16 * (j + 3), 16)] * wp_buf[pl.ds(16 * (j + 3), 16)]
            s = _red16((s0 + s1) + (s2 + s3), jnp.add)
            return jnp.where(lane == r, s, sco_c)

        sco = lax.fori_loop(0, RC, score_row, jnp.zeros((16,), jnp.float32))
        m_c = _red16(sco, jnp.maximum)
        m_new = jnp.maximum(m_run, m_c)
        alpha_v = jnp.exp(m_run - m_new)
        p_v = jnp.exp(sco - m_new)
        l_new = l_run * alpha_v + _red16(p_v, jnp.add)

        ps = [p_v[r] for r in range(RC)]

        def acc_col(jj, _):
            for u in range(4):
                j = jj * 4 + u
                t0 = ps[0] * xbuf[0, pl.ds(16 * j, 16)]
                t1 = ps[1] * xbuf[1, pl.ds(16 * j, 16)]
                t2 = ps[2] * xbuf[2, pl.ds(16 * j, 16)]
                t3 = ps[3] * xbuf[3, pl.ds(16 * j, 16)]
                for r in range(4, RC, 4):
                    t0 = t0 + ps[r] * xbuf[r, pl.ds(16 * j, 16)]
                    t1 = t1 + ps[r + 1] * xbuf[r + 1, pl.ds(16 * j, 16)]
                    t2 = t2 + ps[r + 2] * xbuf[r + 2, pl.ds(16 * j, 16)]
                    t3 = t3 + ps[r + 3] * xbuf[r + 3, pl.ds(16 * j, 16)]
                acc_ref[pl.ds(16 * j, 16)] = (acc_ref[pl.ds(16 * j, 16)] * alpha_v
                                              + ((t0 + t1) + (t2 + t3)))
            return 0

        lax.fori_loop(0, NV // 4, acc_col, 0)
        return (m_new, l_new)

    copy_in(0, xbuf0, sem0).start()

    def pair_body(kk, carry):
        k0 = 2 * kk
        copy_in(k0, xbuf0, sem0).wait()
        copy_in(k0 + 1, xbuf1, sem1).start()
        carry = process(xbuf0, carry)
        copy_in(k0 + 1, xbuf1, sem1).wait()

        @pl.when(k0 + 2 < NCH)
        def _prefetch():
            copy_in(k0 + 2, xbuf0, sem0).start()

        return process(xbuf1, carry)

    m_fin, l_fin = lax.fori_loop(
        0, NCH // 2, pair_body,
        (jnp.full((16,), -jnp.inf, jnp.float32), jnp.zeros((16,), jnp.float32)))

    lane = lax.iota(jnp.int32, 16)
    st_ref[...] = jnp.where(lane == 0, m_fin,
                            jnp.where(lane == 1, l_fin,
                                      jnp.zeros((16,), jnp.float32)))
    pltpu.sync_copy(acc_ref, acc_hbm.at[wid])
    pltpu.sync_copy(st_ref, stats_hbm.at[wid])


def _sc_pool(x, wp_flat):
    return pl.kernel(
        _sc_pool_body,
        out_type=(jax.ShapeDtypeStruct((B * SUB_PER_B, D), jnp.float32),
                  jax.ShapeDtypeStruct((B * SUB_PER_B, 16), jnp.float32)),
        mesh=plsc.VectorSubcoreMesh(core_axis_name="c", subcore_axis_name="s"),
        scratch_types=[
            pltpu.VMEM((RC, D), jnp.float32),
            pltpu.VMEM((RC, D), jnp.float32),
            pltpu.VMEM((D,), jnp.float32),
            pltpu.VMEM((D,), jnp.float32),
            pltpu.VMEM((16,), jnp.float32),
            pltpu.SemaphoreType.DMA,
            pltpu.SemaphoreType.DMA,
        ],
    )(x, wp_flat)


def _tc_pool_kernel(x_ref, w_pool_ref, acc_out, m_out, l_out, m_ref, l_ref):
    b = pl.program_id(0)
    c = pl.program_id(1)

    @pl.when(c == 0)
    def _init():
        m_ref[0] = -jnp.inf
        l_ref[0] = 0.0

    x_blk = x_ref[0]
    wp_row = w_pool_ref[...].reshape(1, D)
    s = jnp.sum(x_blk * wp_row, axis=1, keepdims=True)
    m_c = jnp.max(s)
    m_prev = m_ref[0]
    m_new = jnp.maximum(m_prev, m_c)
    m_ref[0] = m_new
    alpha = jnp.exp(m_prev - m_new)
    p = jnp.exp(s - m_new)
    l_ref[0] = l_ref[0] * alpha + jnp.sum(p)
    acc_c = jnp.dot(p.T, x_blk, preferred_element_type=jnp.float32)

    @pl.when(c == 0)
    def _first():
        acc_out[pl.ds(b, 1), :] = acc_c

    @pl.when(c > 0)
    def _rest():
        acc_out[pl.ds(b, 1), :] = acc_out[pl.ds(b, 1), :] * alpha + acc_c

    @pl.when(c == NC - 1)
    def _fin():
        m_out[pl.ds(b, 1), :] = jnp.full((1, 128), m_ref[0], jnp.float32)
        l_out[pl.ds(b, 1), :] = jnp.full((1, 128), l_ref[0], jnp.float32)


def _tc_pool(x, w_pool):
    return pl.pallas_call(
        _tc_pool_kernel,
        grid=(B, NC),
        in_specs=[
            pl.BlockSpec((1, CS, D), lambda b, c: (b, c + SC_ROWS // CS, 0)),
            pl.BlockSpec((D, 1), lambda b, c: (0, 0)),
        ],
        out_specs=[
            pl.BlockSpec((B, D), lambda b, c: (0, 0)),
            pl.BlockSpec((B, 128), lambda b, c: (0, 0)),
            pl.BlockSpec((B, 128), lambda b, c: (0, 0)),
        ],
        out_shape=[
            jax.ShapeDtypeStruct((B, D), jnp.float32),
            jax.ShapeDtypeStruct((B, 128), jnp.float32),
            jax.ShapeDtypeStruct((B, 128), jnp.float32),
        ],
        scratch_shapes=[
            pltpu.SMEM((1,), jnp.float32),
            pltpu.SMEM((1,), jnp.float32),
        ],
    )(x, w_pool)


def _combine_kernel(acc_tc_ref, mtc_ref, ltc_ref, acc_sc_ref, st_sc_ref,
                    w1_ref, b1_ref, w2_ref, b2_ref, out_ref):
    msc = st_sc_ref[...][:, :, 0]
    lsc = st_sc_ref[...][:, :, 1]
    mtc = mtc_ref[...][:, 0:1]
    ltc = ltc_ref[...][:, 0:1]
    m_all = jnp.maximum(jnp.max(msc, axis=1, keepdims=True), mtc)
    sc_scale = jnp.exp(msc - m_all)
    tc_scale = jnp.exp(mtc - m_all)
    l_tot = ltc * tc_scale + jnp.sum(lsc * sc_scale, axis=1, keepdims=True)
    acc_tot = (acc_tc_ref[...] * tc_scale
               + jnp.sum(acc_sc_ref[...] * sc_scale[:, :, None], axis=1))
    pooled = acc_tot / l_tot

    h = jnp.dot(pooled, w1_ref[...], preferred_element_type=jnp.float32)
    h = jnp.maximum(h + b1_ref[...], 0.0)
    logits = jnp.dot(h, w2_ref[...], preferred_element_type=jnp.float32)
    logits = logits + b2_ref[...]

    col = jax.lax.broadcasted_iota(jnp.int32, (B, NUM_OUT), 1)
    m1 = jnp.max(logits, axis=1, keepdims=True)
    i1 = jnp.min(jnp.where(logits == m1, col, NUM_OUT), axis=1, keepdims=True)
    l2 = jnp.where(col == i1, -jnp.inf, logits)
    m2 = jnp.max(l2, axis=1, keepdims=True)
    i2 = jnp.min(jnp.where(l2 == m2, col, NUM_OUT), axis=1, keepdims=True)
    keep = (col == i1) | (col == i2)
    e = jnp.where(keep, jnp.exp(logits - m1), 0.0)
    out_ref[...] = e / jnp.sum(e, axis=1, keepdims=True)


def _combine(acc_tc, m_tc, l_tc, acc_sc, st_sc, w1, b1, w2, b2):
    return pl.pallas_call(
        _combine_kernel,
        out_shape=jax.ShapeDtypeStruct((B, NUM_OUT), jnp.float32),
    )(acc_tc, m_tc, l_tc, acc_sc, st_sc, w1, b1, w2, b2)


@jax.jit
def kernel(x, w_pool, b_pool, w1, b1, w2, b2):
    del b_pool
    wp_flat = w_pool.reshape(D)
    acc_sc, st_sc = _sc_pool(x, wp_flat)
    acc_tc, m_tc, l_tc = _tc_pool(x, w_pool)
    return _combine(acc_tc, m_tc, l_tc,
                    acc_sc.reshape(B, SUB_PER_B, D),
                    st_sc.reshape(B, SUB_PER_B, 16),
                    w1, b1.reshape(1, HID), w2, b2.reshape(1, NUM_OUT))


# speedup vs baseline: 1.8531x; 1.3531x over previous
import functools

import jax
import jax.numpy as jnp
from jax.experimental import pallas as pl
from jax.experimental.pallas import tpu as pltpu

B, S, D = 4, 8192, 1024
HID = 512
NUM_OUT = 8

SUB = 1024
PER_B = S // SUB
NSUB = B * PER_B
NBUF = 8
INFLIGHT = 4
G = 2
NGRP = NSUB // G
GPB = PER_B // G


def _router_kernel(x_ref, wp_ref, w1_ref, b1_ref, w2_ref, b2_ref,
                   out_ref, bufs, sems):
    def dma(i):
        return pltpu.make_async_copy(
            x_ref.at[i // PER_B, pl.ds((i % PER_B) * SUB, SUB), :],
            bufs.at[i % NBUF],
            sems.at[i % NBUF])

    for i in range(INFLIGHT):
        dma(i).start()

    wp_row = wp_ref[...].reshape(1, D)
    pooled_rows = []
    m = jnp.float32(-jnp.inf)
    l = jnp.float32(0.0)
    acc = jnp.zeros((1, D), jnp.float32)

    for g in range(NGRP):
        if g % GPB == 0:
            m = jnp.float32(-jnp.inf)
            l = jnp.float32(0.0)
            acc = jnp.zeros((1, D), jnp.float32)
        xs, ss = [], []
        for u in range(G):
            i = G * g + u
            dma(i).wait()
            if i + INFLIGHT < NSUB:
                dma(i + INFLIGHT).start()
            xb = bufs[i % NBUF]
            xs.append(xb)
            ss.append(jnp.sum(xb * wp_row, axis=1, keepdims=True))
        m_c = jnp.max(ss[0])
        for u in range(1, G):
            m_c = jnp.maximum(m_c, jnp.max(ss[u]))
        m_new = jnp.maximum(m, m_c)
        alpha = jnp.exp(m - m_new)
        ps = [jnp.exp(s - m_new) for s in ss]
        l_c = ps[0].sum()
        acc_c = jnp.dot(ps[0].T, xs[0], preferred_element_type=jnp.float32)
        for u in range(1, G):
            l_c = l_c + ps[u].sum()
            acc_c = acc_c + jnp.dot(ps[u].T, xs[u],
                                    preferred_element_type=jnp.float32)
        m = m_new
        l = l * alpha + l_c
        acc = acc * alpha + acc_c
        if g % GPB == GPB - 1:
            pooled_rows.append(acc / l)

    pooled = jnp.concatenate(pooled_rows, axis=0)
    h = jnp.dot(pooled, w1_ref[...], preferred_element_type=jnp.float32)
    h = jnp.maximum(h + b1_ref[...], 0.0)
    logits = jnp.dot(h, w2_ref[...], preferred_element_type=jnp.float32)
    logits = logits + b2_ref[...]

    col = jax.lax.broadcasted_iota(jnp.int32, (B, NUM_OUT), 1)
    m1 = jnp.max(logits, axis=1, keepdims=True)
    i1 = jnp.min(jnp.where(logits == m1, col, NUM_OUT), axis=1, keepdims=True)
    l2 = jnp.where(col == i1, -jnp.inf, logits)
    m2 = jnp.max(l2, axis=1, keepdims=True)
    i2 = jnp.min(jnp.where(l2 == m2, col, NUM_OUT), axis=1, keepdims=True)
    keep = (col == i1) | (col == i2)
    e = jnp.where(keep, jnp.exp(logits - m1), 0.0)
    out_ref[...] = e / jnp.sum(e, axis=1, keepdims=True)


@jax.jit
def kernel(x, w_pool, b_pool, w1, b1, w2, b2):
    del b_pool
    return pl.pallas_call(
        _router_kernel,
        in_specs=[
            pl.BlockSpec(memory_space=pl.ANY),
            pl.BlockSpec((D, 1), lambda: (0, 0)),
            pl.BlockSpec((D, HID), lambda: (0, 0)),
            pl.BlockSpec((1, HID), lambda: (0, 0)),
            pl.BlockSpec((HID, NUM_OUT), lambda: (0, 0)),
            pl.BlockSpec((1, NUM_OUT), lambda: (0, 0)),
        ],
        out_specs=pl.BlockSpec((B, NUM_OUT), lambda: (0, 0)),
        out_shape=jax.ShapeDtypeStruct((B, NUM_OUT), jnp.float32),
        scratch_shapes=[
            pltpu.VMEM((NBUF, SUB, D), jnp.float32),
            pltpu.SemaphoreType.DMA((NBUF,)),
        ],
    )(x, w_pool, w1, b1.reshape(1, HID), w2, b2.reshape(1, NUM_OUT))
